# baseline (speedup 1.0000x reference)
import functools

import jax
import jax.numpy as jnp
from jax import lax
from jax.experimental import pallas as pl
from jax.experimental.pallas import tpu as pltpu
from jax.experimental.pallas import tpu_sc as plsc

LOG_EPS = 1e-8
NE = 320000
DIM = 128
LANES = 16
NWORK = 32
CH = 128
NCH_SIGN = NE // CH
NCH_ALL = 2 * NCH_SIGN
W32 = DIM // 2
OROW = CH * LANES // DIM
TROW = 2 * NE * LANES // DIM


def _sc_partials(z, pos_edge_index, neg_edge_index):
  mesh = plsc.VectorSubcoreMesh(core_axis_name="c", subcore_axis_name="s")

  @functools.partial(
      pl.kernel,
      out_type=jax.ShapeDtypeStruct((TROW, DIM), jnp.float32),
      compiler_params=pltpu.CompilerParams(use_tc_tiling_on_sc=False),
      mesh=mesh,
      scratch_types=[
          pltpu.VMEM((CH,), jnp.int32),
          pltpu.VMEM((CH,), jnp.int32),
          pltpu.VMEM((CH,), jnp.int32),
          pltpu.VMEM((CH,), jnp.int32),
          pltpu.VMEM((CH, W32), jnp.uint32),
          pltpu.VMEM((CH, W32), jnp.uint32),
          pltpu.VMEM((CH, W32), jnp.uint32),
          pltpu.VMEM((CH, W32), jnp.uint32),
          pltpu.VMEM((OROW, DIM), jnp.float32),
          pltpu.VMEM((OROW, DIM), jnp.float32),
          pltpu.SemaphoreType.DMA,
          pltpu.SemaphoreType.DMA,
          pltpu.SemaphoreType.DMA,
          pltpu.SemaphoreType.DMA,
          pltpu.SemaphoreType.DMA,
          pltpu.SemaphoreType.DMA,
      ],
  )
  def k(z_hbm, pos_hbm, neg_hbm, out_hbm, si0, di0, si1, di1, sr0, dr0, sr1,
        dr1, ov0, ov1, is0, is1, gs0, gs1, os0, os1):
    wid = lax.axis_index("s") * 2 + lax.axis_index("c")
    nch = NCH_ALL // NWORK + jnp.where(wid < NCH_ALL % NWORK, 1, 0)
    sidx = (si0, si1)
    didx = (di0, di1)
    srows = (sr0, sr1)
    drows = (dr0, dr1)
    outv = (ov0, ov1)
    isem = (is0, is1)
    gsem = (gs0, gs1)
    osem = (os0, os1)

    def issue_idx(i, b):
      c = wid + i * NWORK

      @pl.when(c < NCH_SIGN)
      def _():
        off = pl.multiple_of(c * CH, CH)
        pltpu.async_copy(pos_hbm.at[0, pl.ds(off, CH)], sidx[b], isem[b])
        pltpu.async_copy(pos_hbm.at[1, pl.ds(off, CH)], didx[b], isem[b])

      @pl.when(c >= NCH_SIGN)
      def _():
        off = pl.multiple_of((c - NCH_SIGN) * CH, CH)
        pltpu.async_copy(neg_hbm.at[0, pl.ds(off, CH)], sidx[b], isem[b])
        pltpu.async_copy(neg_hbm.at[1, pl.ds(off, CH)], didx[b], isem[b])

    def wait_idx(b):
      pltpu.make_async_copy(pos_hbm.at[0, pl.ds(0, CH)], sidx[b],
                            isem[b]).wait()
      pltpu.make_async_copy(pos_hbm.at[1, pl.ds(0, CH)], didx[b],
                            isem[b]).wait()

    def issue_gather(b):
      pltpu.async_copy(z_hbm.at[sidx[b]], srows[b], gsem[b])
      pltpu.async_copy(z_hbm.at[didx[b]], drows[b], gsem[b])

    def wait_gather(b):
      pltpu.make_async_copy(z_hbm.at[sidx[b]], srows[b], gsem[b]).wait()
      pltpu.make_async_copy(z_hbm.at[didx[b]], drows[b], gsem[b]).wait()

    def issue_out(i, b):
      c = wid + i * NWORK
      rb = pl.multiple_of(c * OROW, OROW)
      pltpu.async_copy(outv[b], out_hbm.at[pl.ds(rb, OROW)], osem[b])

    def wait_out(b):
      pltpu.make_async_copy(outv[b], out_hbm.at[pl.ds(0, OROW)],
                            osem[b]).wait()

    def compute(b):
      sr, dr, ov = srows[b], drows[b], outv[b]

      def rbody(r, carry):
        for q in range(DIM // LANES):
          e = r * 8 + q
          acc = None
          for j in range(W32 // LANES):
            sv = sr[e, j * LANES:(j + 1) * LANES]
            dv = dr[e, j * LANES:(j + 1) * LANES]
            s0 = lax.bitcast_convert_type(sv << 16, jnp.float32)
            s1 = lax.bitcast_convert_type(sv, jnp.float32)
            d0 = lax.bitcast_convert_type(dv << 16, jnp.float32)
            d1 = lax.bitcast_convert_type(dv, jnp.float32)
            term = s0 * d0 + s1 * d1
            acc = term if acc is None else acc + term
          ov[r, q * LANES:(q + 1) * LANES] = acc
        return carry

      lax.fori_loop(0, OROW, rbody, 0, unroll=2)

    issue_idx(0, 0)
    issue_idx(1, 1)
    wait_idx(0)
    issue_gather(0)

    def step(i, b):
      @pl.when(i < nch)
      def _():
        wait_gather(b)

        @pl.when(i + 1 < nch)
        def _():
          wait_idx(1 - b)
          issue_gather(1 - b)

        @pl.when(i + 2 < nch)
        def _():
          issue_idx(i + 2, b)

        @pl.when(i >= 2)
        def _():
          wait_out(b)

        compute(b)
        issue_out(i, b)

    def jbody(j, carry):
      step(2 * j, 0)
      step(2 * j + 1, 1)
      return carry

    lax.fori_loop(0, (NCH_ALL // NWORK + 1 + 1) // 2, jbody, 0)
    wait_out(0)
    wait_out(1)

  return k(z, pos_edge_index, neg_edge_index)


def _loss_body(p_ref, o_ref, *, npos_blocks):
  i = pl.program_id(0)
  x = p_ref[...]
  r = lax.broadcasted_iota(jnp.int32, (DIM, DIM // LANES), 0)
  c = lax.broadcasted_iota(jnp.int32, (DIM, DIM // LANES), 1)
  fold = jnp.where(r // LANES == c, 1.0, 0.0).astype(jnp.float32)
  v = jnp.dot(x, fold, preferred_element_type=jnp.float32,
              precision=lax.Precision.HIGHEST)
  s = jax.nn.sigmoid(v)
  arg = jnp.where(i < npos_blocks, s, 1.0 - s) + LOG_EPS
  t = -jnp.log(arg)

  @pl.when(i == 0)
  def _():
    o_ref[...] = jnp.zeros((1, 1), jnp.float32)

  o_ref[...] += jnp.full((1, 1), jnp.sum(t) * (1.0 / NE), jnp.float32)


def _tc_loss(partials):
  rows = 4000
  grid = TROW // rows
  body = functools.partial(_loss_body, npos_blocks=grid // 2)
  out = pl.pallas_call(
      body,
      out_shape=jax.ShapeDtypeStruct((1, 1), jnp.float32),
      grid=(grid,),
      in_specs=[pl.BlockSpec((rows, DIM), lambda i: (i, 0))],
      out_specs=pl.BlockSpec((1, 1), lambda i: (0, 0)),
  )(partials)
  return out[0, 0]


def _pack_body(z_ref, o_ref):
  t = lax.bitcast_convert_type(z_ref[...], jnp.uint32)
  bf = (t + jnp.uint32(0x7FFF) + ((t >> 16) & jnp.uint32(1))) >> 16
  bff = bf.astype(jnp.float32)
  r = lax.broadcasted_iota(jnp.int32, (DIM, W32), 0)
  c = lax.broadcasted_iota(jnp.int32, (DIM, W32), 1)
  ev_m = (r == 2 * c).astype(jnp.float32)
  od_m = (r == 2 * c + 1).astype(jnp.float32)
  ev = jnp.dot(bff, ev_m, preferred_element_type=jnp.float32,
               precision=lax.Precision.HIGHEST).astype(jnp.uint32)
  od = jnp.dot(bff, od_m, preferred_element_type=jnp.float32,
               precision=lax.Precision.HIGHEST).astype(jnp.uint32)
  o_ref[...] = ev | (od << 16)


def _pack_z(z):
  return pl.pallas_call(
      _pack_body,
      out_shape=jax.ShapeDtypeStruct((z.shape[0], W32), jnp.uint32),
  )(z)


def kernel(z, pos_edge_index, neg_edge_index):
  partials = _sc_partials(_pack_z(z), pos_edge_index, neg_edge_index)
  return _tc_loss(partials)


# speedup vs baseline: 1.0720x; 1.0720x over previous
import functools

import jax
import jax.numpy as jnp
from jax import lax
from jax.experimental import pallas as pl
from jax.experimental.pallas import tpu as pltpu
from jax.experimental.pallas import tpu_sc as plsc

LOG_EPS = 1e-8
NE = 320000
DIM = 128
LANES = 16
NWORK = 32
CH = 128
NCH_SIGN = NE // CH
NCH_ALL = 2 * NCH_SIGN
W32 = DIM // 2
OROW = CH * LANES // DIM
TROW = 2 * NE * LANES // DIM


def _sc_partials(z, pos_edge_index, neg_edge_index):
  mesh = plsc.VectorSubcoreMesh(core_axis_name="c", subcore_axis_name="s")

  @functools.partial(
      pl.kernel,
      out_type=jax.ShapeDtypeStruct((TROW, DIM), jnp.float32),
      compiler_params=pltpu.CompilerParams(use_tc_tiling_on_sc=False),
      mesh=mesh,
      scratch_types=[
          pltpu.VMEM((CH,), jnp.int32),
          pltpu.VMEM((CH,), jnp.int32),
          pltpu.VMEM((CH,), jnp.int32),
          pltpu.VMEM((CH,), jnp.int32),
          pltpu.VMEM((CH, W32), jnp.uint32),
          pltpu.VMEM((CH, W32), jnp.uint32),
          pltpu.VMEM((CH, W32), jnp.uint32),
          pltpu.VMEM((CH, W32), jnp.uint32),
          pltpu.VMEM((OROW, DIM), jnp.float32),
          pltpu.VMEM((OROW, DIM), jnp.float32),
          pltpu.SemaphoreType.DMA,
          pltpu.SemaphoreType.DMA,
          pltpu.SemaphoreType.DMA,
          pltpu.SemaphoreType.DMA,
          pltpu.SemaphoreType.DMA,
          pltpu.SemaphoreType.DMA,
      ],
  )
  def k(z_hbm, pos_hbm, neg_hbm, out_hbm, si0, di0, si1, di1, sr0, dr0, sr1,
        dr1, ov0, ov1, is0, is1, gs0, gs1, os0, os1):
    wid = lax.axis_index("s") * 2 + lax.axis_index("c")
    nch = NCH_ALL // NWORK + jnp.where(wid < NCH_ALL % NWORK, 1, 0)
    sidx = (si0, si1)
    didx = (di0, di1)
    srows = (sr0, sr1)
    drows = (dr0, dr1)
    outv = (ov0, ov1)
    isem = (is0, is1)
    gsem = (gs0, gs1)
    osem = (os0, os1)

    def issue_idx(i, b):
      c = wid + i * NWORK

      @pl.when(c < NCH_SIGN)
      def _():
        off = pl.multiple_of(c * CH, CH)
        pltpu.async_copy(pos_hbm.at[0, pl.ds(off, CH)], sidx[b], isem[b])
        pltpu.async_copy(pos_hbm.at[1, pl.ds(off, CH)], didx[b], isem[b])

      @pl.when(c >= NCH_SIGN)
      def _():
        off = pl.multiple_of((c - NCH_SIGN) * CH, CH)
        pltpu.async_copy(neg_hbm.at[0, pl.ds(off, CH)], sidx[b], isem[b])
        pltpu.async_copy(neg_hbm.at[1, pl.ds(off, CH)], didx[b], isem[b])

    def wait_idx(b):
      pltpu.make_async_copy(pos_hbm.at[0, pl.ds(0, CH)], sidx[b],
                            isem[b]).wait()
      pltpu.make_async_copy(pos_hbm.at[1, pl.ds(0, CH)], didx[b],
                            isem[b]).wait()

    def issue_gather(b):
      pltpu.async_copy(z_hbm.at[sidx[b]], srows[b], gsem[b])
      pltpu.async_copy(z_hbm.at[didx[b]], drows[b], gsem[b])

    def wait_gather(b):
      pltpu.make_async_copy(z_hbm.at[sidx[b]], srows[b], gsem[b]).wait()
      pltpu.make_async_copy(z_hbm.at[didx[b]], drows[b], gsem[b]).wait()

    def issue_out(i, b):
      c = wid + i * NWORK
      rb = pl.multiple_of(c * OROW, OROW)
      pltpu.async_copy(outv[b], out_hbm.at[pl.ds(rb, OROW)], osem[b])

    def wait_out(b):
      pltpu.make_async_copy(outv[b], out_hbm.at[pl.ds(0, OROW)],
                            osem[b]).wait()

    def compute(b):
      sr, dr, ov = srows[b], drows[b], outv[b]

      def rbody(r, carry):
        for q in range(DIM // LANES):
          e = r * 8 + q
          acc = None
          for j in range(W32 // LANES):
            sv = sr[e, j * LANES:(j + 1) * LANES]
            dv = dr[e, j * LANES:(j + 1) * LANES]
            s0 = lax.bitcast_convert_type(sv << 16, jnp.float32)
            s1 = lax.bitcast_convert_type(sv, jnp.float32)
            d0 = lax.bitcast_convert_type(dv << 16, jnp.float32)
            d1 = lax.bitcast_convert_type(dv, jnp.float32)
            term = s0 * d0 + s1 * d1
            acc = term if acc is None else acc + term
          ov[r, q * LANES:(q + 1) * LANES] = acc
        return carry

      lax.fori_loop(0, OROW, rbody, 0, unroll=2)

    issue_idx(0, 0)
    issue_idx(1, 1)
    wait_idx(0)
    issue_gather(0)

    def step(i, b):
      @pl.when(i < nch)
      def _():
        wait_gather(b)

        @pl.when(i + 1 < nch)
        def _():
          wait_idx(1 - b)
          issue_gather(1 - b)

        @pl.when(i + 2 < nch)
        def _():
          issue_idx(i + 2, b)

        @pl.when(i >= 2)
        def _():
          wait_out(b)

        compute(b)
        issue_out(i, b)

    def jbody(j, carry):
      step(2 * j, 0)
      step(2 * j + 1, 1)
      return carry

    lax.fori_loop(0, (NCH_ALL // NWORK + 1 + 1) // 2, jbody, 0)
    wait_out(0)
    wait_out(1)

  return k(z, pos_edge_index, neg_edge_index)


def _loss_body(p_ref, o_ref, *, npos_blocks):
  i = pl.program_id(0)
  x = p_ref[...]
  r = lax.broadcasted_iota(jnp.int32, (DIM, DIM // LANES), 0)
  c = lax.broadcasted_iota(jnp.int32, (DIM, DIM // LANES), 1)
  fold = jnp.where(r // LANES == c, 1.0, 0.0).astype(jnp.float32)
  v = jnp.dot(x, fold, preferred_element_type=jnp.float32)
  s = jax.nn.sigmoid(v)
  arg = jnp.where(i < npos_blocks, s, 1.0 - s) + LOG_EPS
  t = -jnp.log(arg)

  @pl.when(i == 0)
  def _():
    o_ref[...] = jnp.zeros((1, 1), jnp.float32)

  o_ref[...] += jnp.full((1, 1), jnp.sum(t) * (1.0 / NE), jnp.float32)


def _tc_loss(partials):
  rows = 4000
  grid = TROW // rows
  body = functools.partial(_loss_body, npos_blocks=grid // 2)
  out = pl.pallas_call(
      body,
      out_shape=jax.ShapeDtypeStruct((1, 1), jnp.float32),
      grid=(grid,),
      in_specs=[pl.BlockSpec((rows, DIM), lambda i: (i, 0))],
      out_specs=pl.BlockSpec((1, 1), lambda i: (0, 0)),
  )(partials)
  return out[0, 0]


def _pack_body(z_ref, o_ref):
  t = lax.bitcast_convert_type(z_ref[...], jnp.uint32)
  bf = (t + jnp.uint32(0x7FFF) + ((t >> 16) & jnp.uint32(1))) >> 16
  bff = bf.astype(jnp.float32)
  r = lax.broadcasted_iota(jnp.int32, (DIM, W32), 0)
  c = lax.broadcasted_iota(jnp.int32, (DIM, W32), 1)
  ev_m = (r == 2 * c).astype(jnp.float32)
  od_m = (r == 2 * c + 1).astype(jnp.float32)
  ev = jnp.dot(bff, ev_m, preferred_element_type=jnp.float32,
               precision=lax.Precision.HIGHEST).astype(jnp.uint32)
  od = jnp.dot(bff, od_m, preferred_element_type=jnp.float32,
               precision=lax.Precision.HIGHEST).astype(jnp.uint32)
  o_ref[...] = ev | (od << 16)


def _pack_z(z):
  return pl.pallas_call(
      _pack_body,
      out_shape=jax.ShapeDtypeStruct((z.shape[0], W32), jnp.uint32),
  )(z)


def kernel(z, pos_edge_index, neg_edge_index):
  partials = _sc_partials(_pack_z(z), pos_edge_index, neg_edge_index)
  return _tc_loss(partials)
